# baseline (speedup 1.0000x reference)
import functools
import math

import jax
import jax.numpy as jnp
from jax import lax
from jax.experimental import pallas as pl
from jax.experimental.pallas import tpu as pltpu
from jax.experimental.pallas import tpu_sc as plsc

DIM = 64
LANES = 16


@functools.lru_cache(maxsize=None)
def _make_kernel(BA, T, CB):
    B = BA * T
    C = CB * T
    info = plsc.get_sparse_core_info()
    num_workers = info.num_cores * info.num_subcores
    per_w_batches = BA // num_workers
    n_chunks = per_w_batches // CB
    scale = math.sqrt(DIM)
    mesh = plsc.VectorSubcoreMesh(core_axis_name="c", subcore_axis_name="s")

    @functools.partial(
        pl.kernel,
        mesh=mesh,
        out_type=jax.ShapeDtypeStruct((BA, T, DIM), jnp.float32),
        scratch_types=[
            pltpu.VMEM((C,), jnp.int32),
            pltpu.VMEM((C, DIM), jnp.float32),
            pltpu.SemaphoreType.DMA,
        ],
        compiler_params=pltpu.CompilerParams(use_tc_tiling_on_sc=False),
    )
    def k(idx_hbm, table_hbm, out_hbm, idx_v, rows_v, sem):
        wid = lax.axis_index("s") * info.num_cores + lax.axis_index("c")
        base_b = wid * per_w_batches

        def chunk_body(g, carry):
            b0 = base_b + g * CB
            pltpu.sync_copy(idx_hbm.at[pl.ds(b0 * T, C)], idx_v)
            pltpu.async_copy(table_hbm.at[idx_v], rows_v, sem).wait()

            def row_body(i, c2):
                for j in range(DIM // LANES):
                    s = pl.ds(j * LANES, LANES)
                    rows_v[i, s] = rows_v[i, s] * scale
                return c2

            lax.fori_loop(0, C, row_body, 0)
            for r in range(CB):
                pltpu.sync_copy(
                    rows_v.at[pl.ds(r * T, T)], out_hbm.at[b0 + r]
                )
            return carry

        lax.fori_loop(0, n_chunks, chunk_body, 0)

    return k


def kernel(inp, table):
    b, t = inp.shape
    flat = inp.reshape(b * t).astype(jnp.int32)
    return _make_kernel(b, t, 4)(flat, table)


# speedup vs baseline: 1.0940x; 1.0940x over previous
import functools
import math

import jax
import jax.numpy as jnp
from jax import lax
from jax.experimental import pallas as pl
from jax.experimental.pallas import tpu as pltpu
from jax.experimental.pallas import tpu_sc as plsc

DIM = 64
PADW = 128
LANES = 16


@functools.lru_cache(maxsize=None)
def _make_kernel(BA, T, CB):
    B = BA * T
    C = CB * T
    info = plsc.get_sparse_core_info()
    num_workers = info.num_cores * info.num_subcores
    per_w_batches = BA // num_workers
    n_chunks = per_w_batches // CB
    scale = math.sqrt(DIM)
    mesh = plsc.VectorSubcoreMesh(core_axis_name="c", subcore_axis_name="s")

    @functools.partial(
        pl.kernel,
        mesh=mesh,
        out_type=jax.ShapeDtypeStruct((BA, T, DIM), jnp.float32),
        scratch_types=[
            pltpu.VMEM((C,), jnp.int32),
            pltpu.VMEM((C, PADW), jnp.float32),
            pltpu.VMEM((CB, T, DIM), jnp.float32),
            pltpu.SemaphoreType.DMA,
        ],
        compiler_params=pltpu.CompilerParams(use_tc_tiling_on_sc=True),
    )
    def k(idx_hbm, table_hbm, out_hbm, idx_v, rows_v, slab_v, sem):
        wid = lax.axis_index("s") * info.num_cores + lax.axis_index("c")
        base_b = wid * per_w_batches

        def chunk_body(g, carry):
            b0 = base_b + g * CB
            pltpu.sync_copy(idx_hbm.at[pl.ds(b0 * T, C)], idx_v)
            pltpu.async_copy(table_hbm.at[idx_v], rows_v, sem).wait()

            def t_body(t, r):
                for j in range(DIM // LANES):
                    s = pl.ds(j * LANES, LANES)
                    slab_v[r, t, s] = rows_v[r * T + t, s] * scale
                return r

            def b_body(r, c2):
                lax.fori_loop(0, T, t_body, r)
                return c2

            lax.fori_loop(0, CB, b_body, 0)
            pltpu.sync_copy(slab_v, out_hbm.at[pl.ds(b0, CB)])
            return carry

        lax.fori_loop(0, n_chunks, chunk_body, 0)

    return k


def kernel(inp, table):
    b, t = inp.shape
    flat = inp.reshape(b * t).astype(jnp.int32)
    padded = jnp.pad(table, ((0, 0), (0, PADW - DIM)))
    return _make_kernel(b, t, 2)(flat, padded)
